# baseline (speedup 1.0000x reference)
import functools

import jax
import jax.numpy as jnp
from jax import lax
from jax.experimental import pallas as pl
from jax.experimental.pallas import tpu as pltpu
from jax.experimental.pallas import tpu_sc as plsc

N = 10000
D = 128
E = 320000

NC = 2
NS = 16
NW = NC * NS
CHUNK = 128
NCHK = E // CHUNK
CPW = 80
GRP = 4
GPT = CPW // GRP
NB = GPT // 2
TAIL_W = NW - 1
TAIL_NC = NCHK - TAIL_W * CPW
LAST_G = NCHK // GRP - 1
N_PAD = 10240
RPT = N_PAD // NS

ROW_BLK = 512


EPW = CPW * CHUNK
TAIL_E = E - TAIL_W * EPW


def _deg_body(ei_hbm, zeros_hbm, out_hbm, cnt, idx_v):
    c = lax.axis_index("c")
    s = lax.axis_index("s")
    wid = s * NC + c
    pltpu.sync_copy(zeros_hbm, cnt)

    @pl.when(wid < TAIL_W)
    def _():
        pltpu.sync_copy(ei_hbm.at[0, pl.ds(wid * EPW, EPW)], idx_v)

    @pl.when(wid == TAIL_W)
    def _():
        pltpu.sync_copy(ei_hbm.at[0, pl.ds(TAIL_W * EPW, TAIL_E)],
                        idx_v.at[pl.ds(0, TAIL_E)])

    ones = jnp.ones((16,), jnp.float32)
    nc = jnp.where(wid == TAIL_W, TAIL_E // CHUNK, CPW)

    def chunk(ci, carry):
        for j in range(CHUNK // 16):
            iv = idx_v[pl.ds(ci * CHUNK + j * 16, 16)]
            plsc.addupdate_scatter(cnt, [iv], ones)
        return carry

    lax.fori_loop(0, nc, chunk, 0)
    pltpu.sync_copy(cnt, out_hbm.at[wid])


@jax.jit
def _deg_call(edge_index, zeros1d):
    mesh = plsc.VectorSubcoreMesh(core_axis_name="c", subcore_axis_name="s")
    return pl.kernel(
        _deg_body,
        out_type=jax.ShapeDtypeStruct((NW, N_PAD), jnp.float32),
        mesh=mesh,
        scratch_types=[
            pltpu.VMEM((N_PAD,), jnp.float32),
            pltpu.VMEM((EPW,), jnp.int32),
        ],
        compiler_params=pltpu.CompilerParams(needs_layout_passes=False),
    )(edge_index, zeros1d)


def _msg_body(ei_hbm, g_hbm, zeros_hbm, out_hbm,
              acc, ibA_r, ibA_c, ibB_r, ibB_c, msg0, msg1,
              sem0, sem1, semA, semB):
    c = lax.axis_index("c")
    s = lax.axis_index("s")
    wid = s * NC + c
    pltpu.sync_copy(zeros_hbm, acc.at[pl.ds(s * RPT, RPT)])
    plsc.subcore_barrier()

    def start(cref, buf, sem):
        pltpu.async_copy(g_hbm.at[cref], buf, sem)

    def wait(buf, sem):
        pltpu.make_async_copy(g_hbm.at[ibA_c.at[0]], buf, sem).wait()

    def scatter(rref, buf):
        pltpu.sync_copy(buf, acc.at[rref], add=True)

    def fetch_idx(g, rbuf, cbuf, sem):
        gg = jnp.minimum(wid * GPT + g, LAST_G)
        pltpu.async_copy(ei_hbm.at[0, pl.ds(gg * GRP, GRP)], rbuf, sem)
        pltpu.async_copy(ei_hbm.at[1, pl.ds(gg * GRP, GRP)], cbuf, sem)

    def wait_idx(rbuf, cbuf, sem):
        pltpu.make_async_copy(ei_hbm.at[0, pl.ds(0, GRP)], rbuf, sem).wait()
        pltpu.make_async_copy(ei_hbm.at[1, pl.ds(0, GRP)], cbuf, sem).wait()

    fetch_idx(0, ibA_r, ibA_c, semA)
    wait_idx(ibA_r, ibA_c, semA)
    fetch_idx(1, ibB_r, ibB_c, semB)
    start(ibA_c.at[0], msg0, sem0)

    def body(i, carry):
        ga = lax.rem(2 * i + 2, GPT)
        gb = lax.rem(2 * i + 3, GPT)
        start(ibA_c.at[1], msg1, sem1)
        wait(msg0, sem0)
        scatter(ibA_r.at[0], msg0)
        start(ibA_c.at[2], msg0, sem0)
        wait(msg1, sem1)
        scatter(ibA_r.at[1], msg1)
        start(ibA_c.at[3], msg1, sem1)
        wait(msg0, sem0)
        scatter(ibA_r.at[2], msg0)
        wait_idx(ibB_r, ibB_c, semB)
        start(ibB_c.at[0], msg0, sem0)
        wait(msg1, sem1)
        scatter(ibA_r.at[3], msg1)
        fetch_idx(ga, ibA_r, ibA_c, semA)
        start(ibB_c.at[1], msg1, sem1)
        wait(msg0, sem0)
        scatter(ibB_r.at[0], msg0)
        start(ibB_c.at[2], msg0, sem0)
        wait(msg1, sem1)
        scatter(ibB_r.at[1], msg1)
        start(ibB_c.at[3], msg1, sem1)
        wait(msg0, sem0)
        scatter(ibB_r.at[2], msg0)
        wait_idx(ibA_r, ibA_c, semA)
        start(ibA_c.at[0], msg0, sem0)
        wait(msg1, sem1)
        scatter(ibB_r.at[3], msg1)
        fetch_idx(gb, ibB_r, ibB_c, semB)
        return carry

    nb = jnp.where(wid == TAIL_W, TAIL_NC // (2 * GRP), NB)
    lax.fori_loop(0, nb, body, 0)
    wait(msg0, sem0)
    wait_idx(ibB_r, ibB_c, semB)

    @pl.when(wid == TAIL_W)
    def _():
        pltpu.sync_copy(ei_hbm.at[0, pl.ds(LAST_G * GRP, GRP)], ibA_r)
        pltpu.sync_copy(ei_hbm.at[1, pl.ds(LAST_G * GRP, GRP)], ibA_c)
        start(ibA_c.at[0], msg0, sem0)
        start(ibA_c.at[1], msg1, sem1)
        wait(msg0, sem0)
        scatter(ibA_r.at[0], msg0)
        start(ibA_c.at[2], msg0, sem0)
        wait(msg1, sem1)
        scatter(ibA_r.at[1], msg1)
        start(ibA_c.at[3], msg1, sem1)
        wait(msg0, sem0)
        scatter(ibA_r.at[2], msg0)
        wait(msg1, sem1)
        scatter(ibA_r.at[3], msg1)

    plsc.subcore_barrier()
    pltpu.sync_copy(acc.at[pl.ds(s * RPT, RPT)],
                    out_hbm.at[c, pl.ds(s * RPT, RPT)])


@jax.jit
def _msg_call(ei3, g, zeros128):
    mesh = plsc.VectorSubcoreMesh(core_axis_name="c", subcore_axis_name="s")
    return pl.kernel(
        _msg_body,
        out_type=jax.ShapeDtypeStruct((NC, N_PAD, D), jnp.float32),
        mesh=mesh,
        scratch_types=[
            pltpu.VMEM_SHARED((N_PAD, D), jnp.float32),
            pltpu.VMEM((GRP, CHUNK), jnp.int32),
            pltpu.VMEM((GRP, CHUNK), jnp.int32),
            pltpu.VMEM((GRP, CHUNK), jnp.int32),
            pltpu.VMEM((GRP, CHUNK), jnp.int32),
            pltpu.VMEM((CHUNK, D), jnp.float32),
            pltpu.VMEM((CHUNK, D), jnp.float32),
            pltpu.SemaphoreType.DMA,
            pltpu.SemaphoreType.DMA,
            pltpu.SemaphoreType.DMA,
            pltpu.SemaphoreType.DMA,
        ],
    )(ei3, g, zeros128)


def _g_body(x_ref, w_ref, b_ref, degp_ref, g_ref):
    h = lax.dot_general(x_ref[...], w_ref[...], (((1,), (1,)), ((), ())),
                        preferred_element_type=jnp.float32) + b_ref[...]
    deg = jnp.sum(degp_ref[...], axis=0) + 1.0
    dis = lax.rsqrt(deg)
    g_ref[...] = h * dis[:, None]


@jax.jit
def _g_call(x, W, b2, degp):
    grid = N_PAD // ROW_BLK
    return pl.pallas_call(
        _g_body,
        grid=(grid,),
        in_specs=[
            pl.BlockSpec((ROW_BLK, D), lambda i: (i, 0)),
            pl.BlockSpec((D, D), lambda i: (0, 0)),
            pl.BlockSpec((1, D), lambda i: (0, 0)),
            pl.BlockSpec((NW, ROW_BLK), lambda i: (0, i)),
        ],
        out_specs=pl.BlockSpec((ROW_BLK, D), lambda i: (i, 0)),
        out_shape=jax.ShapeDtypeStruct((N_PAD, D), jnp.float32),
    )(x, W, b2, degp)


def _out_body(p_ref, g_ref, degp_ref, o_ref):
    deg = jnp.sum(degp_ref[...], axis=0) + 1.0
    dis = lax.rsqrt(deg)
    o_ref[...] = (p_ref[0] + p_ref[1] + g_ref[...]) * dis[:, None]


@jax.jit
def _out_call(p, g, degp):
    grid = N_PAD // ROW_BLK
    return pl.pallas_call(
        _out_body,
        grid=(grid,),
        in_specs=[
            pl.BlockSpec((NC, ROW_BLK, D), lambda i: (0, i, 0)),
            pl.BlockSpec((ROW_BLK, D), lambda i: (i, 0)),
            pl.BlockSpec((NW, ROW_BLK), lambda i: (0, i)),
        ],
        out_specs=pl.BlockSpec((ROW_BLK, D), lambda i: (i, 0)),
        out_shape=jax.ShapeDtypeStruct((N, D), jnp.float32),
    )(p, g, degp)


def kernel(x, edge_index, W, b):
    ei = edge_index.astype(jnp.int32)
    ei3 = ei.reshape(2, NCHK, CHUNK)
    zeros1d = jnp.zeros((N_PAD,), jnp.float32)
    zeros128 = jnp.zeros((RPT, D), jnp.float32)

    degp = _deg_call(ei, zeros1d)
    g = _g_call(x, W, b.reshape(1, D), degp)
    p = _msg_call(ei3, g, zeros128)
    return _out_call(p, g, degp)


# speedup vs baseline: 1.0574x; 1.0574x over previous
import functools

import jax
import jax.numpy as jnp
from jax import lax
from jax.experimental import pallas as pl
from jax.experimental.pallas import tpu as pltpu
from jax.experimental.pallas import tpu_sc as plsc

N = 10000
D = 128
E = 320000

NC = 2
NS = 16
NW = NC * NS
CHUNK = 128
NCHK = E // CHUNK
CPW = 80
GRP = 4
GPT = CPW // GRP
NB = GPT // 2
TAIL_W = NW - 1
TAIL_NC = NCHK - TAIL_W * CPW
LAST_G = NCHK // GRP - 1
N_PAD = 10240
RPT = N_PAD // NS

ROW_BLK = 1024


EPW = CPW * CHUNK
TAIL_E = E - TAIL_W * EPW


def _deg_body(ei_hbm, zeros_hbm, out_hbm, cnt, idx_v):
    c = lax.axis_index("c")
    s = lax.axis_index("s")
    wid = s * NC + c
    pltpu.sync_copy(zeros_hbm, cnt)

    @pl.when(wid < TAIL_W)
    def _():
        pltpu.sync_copy(ei_hbm.at[0, pl.ds(wid * EPW, EPW)], idx_v)

    @pl.when(wid == TAIL_W)
    def _():
        pltpu.sync_copy(ei_hbm.at[0, pl.ds(TAIL_W * EPW, TAIL_E)],
                        idx_v.at[pl.ds(0, TAIL_E)])

    ones = jnp.ones((16,), jnp.float32)
    nc = jnp.where(wid == TAIL_W, TAIL_E // CHUNK, CPW)

    def chunk(ci, carry):
        for j in range(CHUNK // 16):
            iv = idx_v[pl.ds(ci * CHUNK + j * 16, 16)]
            plsc.addupdate_scatter(cnt, [iv], ones)
        return carry

    lax.fori_loop(0, nc, chunk, 0)
    pltpu.sync_copy(cnt, out_hbm.at[wid])


@jax.jit
def _deg_call(edge_index, zeros1d):
    mesh = plsc.VectorSubcoreMesh(core_axis_name="c", subcore_axis_name="s")
    return pl.kernel(
        _deg_body,
        out_type=jax.ShapeDtypeStruct((NW, N_PAD), jnp.float32),
        mesh=mesh,
        scratch_types=[
            pltpu.VMEM((N_PAD,), jnp.float32),
            pltpu.VMEM((EPW,), jnp.int32),
        ],
        compiler_params=pltpu.CompilerParams(needs_layout_passes=False),
    )(edge_index, zeros1d)


def _msg_body(ei_hbm, g_hbm, zeros_hbm, out_hbm,
              acc, ibA_r, ibA_c, ibB_r, ibB_c, msg0, msg1,
              sem0, sem1, semA, semB):
    c = lax.axis_index("c")
    s = lax.axis_index("s")
    wid = s * NC + c
    pltpu.sync_copy(zeros_hbm, acc.at[pl.ds(s * RPT, RPT)])
    plsc.subcore_barrier()

    def start(cref, buf, sem):
        pltpu.async_copy(g_hbm.at[cref], buf, sem)

    def wait(buf, sem):
        pltpu.make_async_copy(g_hbm.at[ibA_c.at[0]], buf, sem).wait()

    def scatter(rref, buf):
        pltpu.sync_copy(buf, acc.at[rref], add=True)

    def fetch_idx(g, rbuf, cbuf, sem):
        gg = jnp.minimum(wid * GPT + g, LAST_G)
        pltpu.async_copy(ei_hbm.at[0, pl.ds(gg * GRP, GRP)], rbuf, sem)
        pltpu.async_copy(ei_hbm.at[1, pl.ds(gg * GRP, GRP)], cbuf, sem)

    def wait_idx(rbuf, cbuf, sem):
        pltpu.make_async_copy(ei_hbm.at[0, pl.ds(0, GRP)], rbuf, sem).wait()
        pltpu.make_async_copy(ei_hbm.at[1, pl.ds(0, GRP)], cbuf, sem).wait()

    fetch_idx(0, ibA_r, ibA_c, semA)
    wait_idx(ibA_r, ibA_c, semA)
    fetch_idx(1, ibB_r, ibB_c, semB)
    start(ibA_c.at[0], msg0, sem0)

    def body(i, carry):
        ga = lax.rem(2 * i + 2, GPT)
        gb = lax.rem(2 * i + 3, GPT)
        start(ibA_c.at[1], msg1, sem1)
        wait(msg0, sem0)
        scatter(ibA_r.at[0], msg0)
        start(ibA_c.at[2], msg0, sem0)
        wait(msg1, sem1)
        scatter(ibA_r.at[1], msg1)
        start(ibA_c.at[3], msg1, sem1)
        wait(msg0, sem0)
        scatter(ibA_r.at[2], msg0)
        wait_idx(ibB_r, ibB_c, semB)
        start(ibB_c.at[0], msg0, sem0)
        wait(msg1, sem1)
        scatter(ibA_r.at[3], msg1)
        fetch_idx(ga, ibA_r, ibA_c, semA)
        start(ibB_c.at[1], msg1, sem1)
        wait(msg0, sem0)
        scatter(ibB_r.at[0], msg0)
        start(ibB_c.at[2], msg0, sem0)
        wait(msg1, sem1)
        scatter(ibB_r.at[1], msg1)
        start(ibB_c.at[3], msg1, sem1)
        wait(msg0, sem0)
        scatter(ibB_r.at[2], msg0)
        wait_idx(ibA_r, ibA_c, semA)
        start(ibA_c.at[0], msg0, sem0)
        wait(msg1, sem1)
        scatter(ibB_r.at[3], msg1)
        fetch_idx(gb, ibB_r, ibB_c, semB)
        return carry

    nb = jnp.where(wid == TAIL_W, TAIL_NC // (2 * GRP), NB)
    lax.fori_loop(0, nb, body, 0)
    wait(msg0, sem0)
    wait_idx(ibB_r, ibB_c, semB)

    @pl.when(wid == TAIL_W)
    def _():
        pltpu.sync_copy(ei_hbm.at[0, pl.ds(LAST_G * GRP, GRP)], ibA_r)
        pltpu.sync_copy(ei_hbm.at[1, pl.ds(LAST_G * GRP, GRP)], ibA_c)
        start(ibA_c.at[0], msg0, sem0)
        start(ibA_c.at[1], msg1, sem1)
        wait(msg0, sem0)
        scatter(ibA_r.at[0], msg0)
        start(ibA_c.at[2], msg0, sem0)
        wait(msg1, sem1)
        scatter(ibA_r.at[1], msg1)
        start(ibA_c.at[3], msg1, sem1)
        wait(msg0, sem0)
        scatter(ibA_r.at[2], msg0)
        wait(msg1, sem1)
        scatter(ibA_r.at[3], msg1)

    plsc.subcore_barrier()
    pltpu.sync_copy(acc.at[pl.ds(s * RPT, RPT)],
                    out_hbm.at[c, pl.ds(s * RPT, RPT)])


@jax.jit
def _msg_call(ei3, g, zeros128):
    mesh = plsc.VectorSubcoreMesh(core_axis_name="c", subcore_axis_name="s")
    return pl.kernel(
        _msg_body,
        out_type=jax.ShapeDtypeStruct((NC, N_PAD, D), jnp.float32),
        mesh=mesh,
        scratch_types=[
            pltpu.VMEM_SHARED((N_PAD, D), jnp.float32),
            pltpu.VMEM((GRP, CHUNK), jnp.int32),
            pltpu.VMEM((GRP, CHUNK), jnp.int32),
            pltpu.VMEM((GRP, CHUNK), jnp.int32),
            pltpu.VMEM((GRP, CHUNK), jnp.int32),
            pltpu.VMEM((CHUNK, D), jnp.float32),
            pltpu.VMEM((CHUNK, D), jnp.float32),
            pltpu.SemaphoreType.DMA,
            pltpu.SemaphoreType.DMA,
            pltpu.SemaphoreType.DMA,
            pltpu.SemaphoreType.DMA,
        ],
    )(ei3, g, zeros128)


def _g_body(x_ref, w_ref, b_ref, degp_ref, g_ref):
    h = lax.dot_general(x_ref[...], w_ref[...], (((1,), (1,)), ((), ())),
                        preferred_element_type=jnp.float32) + b_ref[...]
    deg = jnp.sum(degp_ref[...], axis=0) + 1.0
    dis = lax.rsqrt(deg)
    g_ref[...] = h * dis[:, None]


@jax.jit
def _g_call(x, W, b2, degp):
    grid = N_PAD // ROW_BLK
    return pl.pallas_call(
        _g_body,
        grid=(grid,),
        in_specs=[
            pl.BlockSpec((ROW_BLK, D), lambda i: (i, 0)),
            pl.BlockSpec((D, D), lambda i: (0, 0)),
            pl.BlockSpec((1, D), lambda i: (0, 0)),
            pl.BlockSpec((NW, ROW_BLK), lambda i: (0, i)),
        ],
        out_specs=pl.BlockSpec((ROW_BLK, D), lambda i: (i, 0)),
        out_shape=jax.ShapeDtypeStruct((N_PAD, D), jnp.float32),
    )(x, W, b2, degp)


def _out_body(p_ref, g_ref, degp_ref, o_ref):
    deg = jnp.sum(degp_ref[...], axis=0) + 1.0
    dis = lax.rsqrt(deg)
    o_ref[...] = (p_ref[0] + p_ref[1] + g_ref[...]) * dis[:, None]


@jax.jit
def _out_call(p, g, degp):
    grid = N_PAD // ROW_BLK
    return pl.pallas_call(
        _out_body,
        grid=(grid,),
        in_specs=[
            pl.BlockSpec((NC, ROW_BLK, D), lambda i: (0, i, 0)),
            pl.BlockSpec((ROW_BLK, D), lambda i: (i, 0)),
            pl.BlockSpec((NW, ROW_BLK), lambda i: (0, i)),
        ],
        out_specs=pl.BlockSpec((ROW_BLK, D), lambda i: (i, 0)),
        out_shape=jax.ShapeDtypeStruct((N, D), jnp.float32),
    )(p, g, degp)


def kernel(x, edge_index, W, b):
    ei = edge_index.astype(jnp.int32)
    ei3 = ei.reshape(2, NCHK, CHUNK)
    zeros1d = jnp.zeros((N_PAD,), jnp.float32)
    zeros128 = jnp.zeros((RPT, D), jnp.float32)

    degp = _deg_call(ei, zeros1d)
    g = _g_call(x, W, b.reshape(1, D), degp)
    p = _msg_call(ei3, g, zeros128)
    return _out_call(p, g, degp)


# speedup vs baseline: 1.0814x; 1.0226x over previous
import functools

import jax
import jax.numpy as jnp
from jax import lax
from jax.experimental import pallas as pl
from jax.experimental.pallas import tpu as pltpu
from jax.experimental.pallas import tpu_sc as plsc

N = 10000
D = 128
E = 320000

NC = 2
NS = 16
NW = NC * NS
CHUNK = 100
NCHK = E // CHUNK
CPW = NCHK // NW
GRP = 2
GPT = CPW // GRP
NBODY = 16
N_PAD = 10240
RPT = N_PAD // NS

ROW_BLK = 1024


EPW = 10240
TAIL_E = E - (NW - 1) * EPW


def _deg_body(ei_hbm, zeros_hbm, out_hbm, cnt, idx_v):
    c = lax.axis_index("c")
    s = lax.axis_index("s")
    wid = s * NC + c
    pltpu.sync_copy(zeros_hbm, cnt)

    @pl.when(wid < NW - 1)
    def _():
        pltpu.sync_copy(ei_hbm.at[0, pl.ds(wid * EPW, EPW)], idx_v)

    @pl.when(wid == NW - 1)
    def _():
        pltpu.sync_copy(ei_hbm.at[0, pl.ds((NW - 1) * EPW, TAIL_E)],
                        idx_v.at[pl.ds(0, TAIL_E)])

    ones = jnp.ones((16,), jnp.float32)
    nc = jnp.where(wid == NW - 1, TAIL_E // 128, EPW // 128)

    def chunk(ci, carry):
        for j in range(8):
            iv = idx_v[pl.ds(ci * 128 + j * 16, 16)]
            plsc.addupdate_scatter(cnt, [iv], ones)
        return carry

    lax.fori_loop(0, nc, chunk, 0)
    pltpu.sync_copy(cnt, out_hbm.at[wid])


@jax.jit
def _deg_call(edge_index, zeros1d):
    mesh = plsc.VectorSubcoreMesh(core_axis_name="c", subcore_axis_name="s")
    return pl.kernel(
        _deg_body,
        out_type=jax.ShapeDtypeStruct((NW, N_PAD), jnp.float32),
        mesh=mesh,
        scratch_types=[
            pltpu.VMEM((N_PAD,), jnp.float32),
            pltpu.VMEM((EPW,), jnp.int32),
        ],
        compiler_params=pltpu.CompilerParams(needs_layout_passes=False),
    )(edge_index, zeros1d)


def _msg_body(ei_hbm, g_hbm, zeros_hbm, out_hbm,
              acc, iAr, iAc, iBr, iBc, iCr, iCc, m0, m1, m2,
              sg0, sg1, sg2, ss0, ss1, ss2, siA, siB, siC):
    c = lax.axis_index("c")
    s = lax.axis_index("s")
    wid = s * NC + c
    pltpu.sync_copy(zeros_hbm, acc.at[pl.ds(s * RPT, RPT)])
    plsc.subcore_barrier()

    M = [(m0, sg0, ss0), (m1, sg1, ss1), (m2, sg2, ss2)]
    SETS = [(iAr, iAc, siA), (iBr, iBc, siB), (iCr, iCc, siC)]

    def start(k, cref):
        buf, sg, _ = M[k % 3]
        pltpu.async_copy(g_hbm.at[cref], buf, sg)

    def wait_g(k):
        buf, sg, _ = M[k % 3]
        pltpu.make_async_copy(g_hbm.at[iAc.at[0]], buf, sg).wait()

    def scat(k, rref):
        buf, _, ss = M[k % 3]
        pltpu.async_copy(buf, acc.at[rref], ss, add=True)

    def wait_scat(k):
        buf, _, ss = M[k % 3]
        pltpu.make_async_copy(buf, acc.at[iAr.at[0]], ss).wait()

    def fetch_idx(setno, glocal):
        rbuf, cbuf, si = SETS[setno]
        gg = wid * GPT + glocal
        pltpu.async_copy(ei_hbm.at[0, pl.ds(gg * GRP, GRP)], rbuf, si)
        pltpu.async_copy(ei_hbm.at[1, pl.ds(gg * GRP, GRP)], cbuf, si)

    def wait_idx(setno):
        rbuf, cbuf, si = SETS[setno]
        pltpu.make_async_copy(ei_hbm.at[0, pl.ds(0, GRP)], rbuf, si).wait()
        pltpu.make_async_copy(ei_hbm.at[1, pl.ds(0, GRP)], cbuf, si).wait()

    def idxrow(k):
        rbuf, cbuf, _ = SETS[(k % 6) // 2]
        return rbuf.at[k % 2], cbuf.at[k % 2]

    def emit_body(i, first):
        for k in range(6):
            if k == 1:
                wait_idx(1)
            if k == 2:
                fetch_idx(2, 3 * i + 2)
            if k == 3:
                wait_idx(2)
            if k == 4:
                fetch_idx(0, 3 * i + 3)
            if k == 5:
                wait_idx(0)
            if not (first and k < 2):
                wait_scat(k + 1)
            _, cref = idxrow(k + 1)
            start(k + 1, cref)
            wait_g(k)
            rref, _ = idxrow(k)
            scat(k, rref)
        fetch_idx(1, 3 * i + 4)

    fetch_idx(0, 0)
    wait_idx(0)
    fetch_idx(1, 1)
    _, c0 = idxrow(0)
    start(0, c0)
    emit_body(0, first=True)

    def body(i, carry):
        emit_body(i, first=False)
        return carry

    lax.fori_loop(1, NBODY, body, 0)

    wait_idx(1)
    for k, (setno, row) in enumerate([(0, 0), (0, 1), (1, 0), (1, 1)]):
        kk = 96 + k
        if k > 0:
            wait_g(kk)
            scat(kk, SETS[setno][0].at[row])
        else:
            wait_g(kk)
            scat(kk, SETS[setno][0].at[row])
        if k < 3:
            nxt = 96 + k + 1
            s2, r2 = [(0, 1), (1, 0), (1, 1)][k]
            wait_scat(nxt)
            start(nxt, SETS[s2][1].at[r2])
    for k in (97, 98, 99):
        wait_scat(k)

    plsc.subcore_barrier()
    pltpu.sync_copy(acc.at[pl.ds(s * RPT, RPT)],
                    out_hbm.at[c, pl.ds(s * RPT, RPT)])


@jax.jit
def _msg_call(ei3, g, zeros128):
    mesh = plsc.VectorSubcoreMesh(core_axis_name="c", subcore_axis_name="s")
    idx = pltpu.VMEM((GRP, CHUNK), jnp.int32)
    msg = pltpu.VMEM((CHUNK, D), jnp.float32)
    sem = pltpu.SemaphoreType.DMA
    return pl.kernel(
        _msg_body,
        out_type=jax.ShapeDtypeStruct((NC, N_PAD, D), jnp.float32),
        mesh=mesh,
        scratch_types=[
            pltpu.VMEM_SHARED((N_PAD, D), jnp.float32),
            idx, idx, idx, idx, idx, idx, msg, msg, msg,
            sem, sem, sem, sem, sem, sem, sem, sem, sem,
        ],
        compiler_params=pltpu.CompilerParams(use_tc_tiling_on_sc=False),
    )(ei3, g, zeros128)


def _g_body(x_ref, w_ref, b_ref, degp_ref, g_ref):
    h = lax.dot_general(x_ref[...], w_ref[...], (((1,), (1,)), ((), ())),
                        preferred_element_type=jnp.float32) + b_ref[...]
    deg = jnp.sum(degp_ref[...], axis=0) + 1.0
    dis = lax.rsqrt(deg)
    g_ref[...] = h * dis[:, None]


@jax.jit
def _g_call(x, W, b2, degp):
    grid = N_PAD // ROW_BLK
    return pl.pallas_call(
        _g_body,
        grid=(grid,),
        in_specs=[
            pl.BlockSpec((ROW_BLK, D), lambda i: (i, 0)),
            pl.BlockSpec((D, D), lambda i: (0, 0)),
            pl.BlockSpec((1, D), lambda i: (0, 0)),
            pl.BlockSpec((NW, ROW_BLK), lambda i: (0, i)),
        ],
        out_specs=pl.BlockSpec((ROW_BLK, D), lambda i: (i, 0)),
        out_shape=jax.ShapeDtypeStruct((N_PAD, D), jnp.float32),
    )(x, W, b2, degp)


def _out_body(p_ref, g_ref, degp_ref, o_ref):
    deg = jnp.sum(degp_ref[...], axis=0) + 1.0
    dis = lax.rsqrt(deg)
    o_ref[...] = (p_ref[0] + p_ref[1] + g_ref[...]) * dis[:, None]


@jax.jit
def _out_call(p, g, degp):
    grid = N_PAD // ROW_BLK
    return pl.pallas_call(
        _out_body,
        grid=(grid,),
        in_specs=[
            pl.BlockSpec((NC, ROW_BLK, D), lambda i: (0, i, 0)),
            pl.BlockSpec((ROW_BLK, D), lambda i: (i, 0)),
            pl.BlockSpec((NW, ROW_BLK), lambda i: (0, i)),
        ],
        out_specs=pl.BlockSpec((ROW_BLK, D), lambda i: (i, 0)),
        out_shape=jax.ShapeDtypeStruct((N, D), jnp.float32),
    )(p, g, degp)


def kernel(x, edge_index, W, b):
    ei = edge_index.astype(jnp.int32)
    ei3 = ei.reshape(2, NCHK, CHUNK)
    zeros1d = jnp.zeros((N_PAD,), jnp.float32)
    zeros128 = jnp.zeros((RPT, D), jnp.float32)

    degp = _deg_call(ei, zeros1d)
    g = _g_call(x, W, b.reshape(1, D), degp)
    p = _msg_call(ei3, g, zeros128)
    return _out_call(p, g, degp)
